# baseline (speedup 1.0000x reference)
import functools

import jax
import jax.numpy as jnp
from jax import lax
from jax.experimental import pallas as pl
from jax.experimental.pallas import tpu as pltpu
from jax.experimental.pallas import tpu_sc as plsc

N_NODES = 10000
N_EDGES = 320000
D_FEAT = 128
NUM_LAYERS = 3

NC = 2
NS = 16
NW = NC * NS
B = 128
NB = 81
E_PAD = NW * NB * B
NBUF = 3
ROWS_PER_TILE = 624
TAIL_ROWS = N_NODES - NS * ROWS_PER_TILE

CROWS = 312

_MESH = plsc.VectorSubcoreMesh(
    core_axis_name="c", subcore_axis_name="s", num_cores=NC)


@functools.partial(
    pl.kernel,
    out_type=jax.ShapeDtypeStruct((NC, N_NODES, D_FEAT), jnp.float32),
    mesh=_MESH,
    scratch_types=[
        pltpu.VMEM_SHARED((N_NODES, D_FEAT), jnp.float32),
        [pltpu.VMEM((B,), jnp.int32)] * NBUF,
        [pltpu.VMEM((B,), jnp.int32)] * NBUF,
        [pltpu.VMEM((B,), jnp.float32)] * NBUF,
        [pltpu.VMEM((B, D_FEAT), jnp.float32)] * NBUF,
        [pltpu.SemaphoreType.DMA] * NBUF,
        [pltpu.SemaphoreType.DMA] * NBUF,
        [pltpu.SemaphoreType.DMA] * NBUF,
    ],
)
def _spmm(h2_hbm, rows_hbm, cols_hbm, vals_hbm, part_hbm,
          accum, colvs, rowvs, valvs, rbufs, isems, gsems, ssems):
    cid = lax.axis_index("c")
    sid = lax.axis_index("s")
    wid = cid * NS + sid
    h_hbm = h2_hbm.at[cid]
    zero16 = jnp.zeros((16,), jnp.float32)

    def issue_idx(b, u):
        base = (wid * NB + b) * B
        pltpu.async_copy(cols_hbm.at[pl.ds(base, B)], colvs[u], isems[u])
        pltpu.async_copy(rows_hbm.at[pl.ds(base, B)], rowvs[u], isems[u])
        pltpu.async_copy(vals_hbm.at[pl.ds(base, B)], valvs[u], isems[u])

    def wait_idx(u):
        pltpu.make_async_copy(cols_hbm.at[pl.ds(0, B)], colvs[u],
                              isems[u]).wait()
        pltpu.make_async_copy(rows_hbm.at[pl.ds(0, B)], rowvs[u],
                              isems[u]).wait()
        pltpu.make_async_copy(vals_hbm.at[pl.ds(0, B)], valvs[u],
                              isems[u]).wait()

    def issue_gather(u):
        pltpu.async_copy(h_hbm.at[colvs[u]], rbufs[u], gsems[u])

    def wait_gather(u):
        pltpu.make_async_copy(h_hbm.at[colvs[u]], rbufs[u], gsems[u]).wait()

    def issue_scatter(u):
        pltpu.async_copy(rbufs[u], accum.at[rowvs[u]], ssems[u], add=True)

    def wait_scatter(u):
        pltpu.make_async_copy(rbufs[u], accum.at[rowvs[u]], ssems[u]).wait()

    def scale(u):
        buf = rbufs[u]
        valv = valvs[u]

        def group_body(g, _):
            vv = valv[pl.ds(g * 16, 16)]
            for t in range(16):
                v = vv[t]
                row = g * 16 + t
                for kk in range(D_FEAT // 16):
                    sl = pl.ds(kk * 16, 16)
                    buf[row, sl] = buf[row, sl] * v
            return 0
        lax.fori_loop(0, B // 16, group_body, 0)

    def zrow(i, _):
        for kk in range(D_FEAT // 16):
            rbufs[0][i, pl.ds(kk * 16, 16)] = zero16
        return 0
    lax.fori_loop(0, B, zrow, 0)
    for off, sz in [(0, B), (B, B), (2 * B, B), (3 * B, B), (4 * B, 112)]:
        r0 = pl.multiple_of(sid * ROWS_PER_TILE + off, 8)
        pltpu.sync_copy(rbufs[0].at[pl.ds(0, sz)], accum.at[pl.ds(r0, sz)])

    @pl.when(sid == NS - 1)
    def _():
        pltpu.sync_copy(rbufs[0].at[pl.ds(0, TAIL_ROWS)],
                        accum.at[pl.ds(NS * ROWS_PER_TILE, TAIL_ROWS)])

    plsc.subcore_barrier()

    for u in range(NBUF):
        issue_idx(u, u)
    for u in range(NBUF):
        wait_idx(u)
        issue_gather(u)

    def iter_body(i, _):
        for u in range(NBUF):
            b = i * NBUF + u
            prev = (u + NBUF - 1) % NBUF
            more = jnp.logical_and(b >= 1, b + NBUF - 1 < NB)
            wait_gather(u)

            @pl.when(more)
            def _():
                wait_scatter(prev)
                issue_idx(b + NBUF - 1, prev)

            scale(u)
            issue_scatter(u)

            @pl.when(more)
            def _():
                wait_idx(prev)
                issue_gather(prev)
        return 0

    lax.fori_loop(0, NB // NBUF, iter_body, 0)
    for u in range(NBUF):
        wait_scatter(u)
    plsc.subcore_barrier()

    for off, sz in [(0, B), (B, B), (2 * B, B), (3 * B, B), (4 * B, 112)]:
        r0 = pl.multiple_of(sid * ROWS_PER_TILE + off, 8)
        pltpu.sync_copy(accum.at[pl.ds(r0, sz)],
                        part_hbm.at[cid].at[pl.ds(r0, sz)])

    @pl.when(sid == NS - 1)
    def _():
        r0 = NS * ROWS_PER_TILE
        pltpu.sync_copy(accum.at[pl.ds(r0, TAIL_ROWS)],
                        part_hbm.at[cid].at[pl.ds(r0, TAIL_ROWS)])


@functools.partial(
    pl.kernel,
    out_type=jax.ShapeDtypeStruct((NC, N_NODES, D_FEAT), jnp.float32),
    mesh=_MESH,
    scratch_types=[
        pltpu.VMEM((CROWS, D_FEAT), jnp.float32),
        pltpu.VMEM((CROWS, D_FEAT), jnp.float32),
    ],
)
def _combine(part_hbm, h2_hbm, bufa, bufb):
    cid = lax.axis_index("c")
    sid = lax.axis_index("s")
    wid = cid * NS + sid
    r0 = pl.multiple_of(wid * CROWS, 8)

    def add_rows(nrows):
        def body(i, _):
            for kk in range(D_FEAT // 16):
                sl = pl.ds(kk * 16, 16)
                bufa[i, sl] = bufa[i, sl] + bufb[i, sl]
            return 0
        lax.fori_loop(0, nrows, body, 0)

    pltpu.sync_copy(part_hbm.at[0].at[pl.ds(r0, CROWS)], bufa)
    pltpu.sync_copy(part_hbm.at[1].at[pl.ds(r0, CROWS)], bufb)
    add_rows(CROWS)
    pltpu.sync_copy(bufa, h2_hbm.at[0].at[pl.ds(r0, CROWS)])
    pltpu.sync_copy(bufa, h2_hbm.at[1].at[pl.ds(r0, CROWS)])

    @pl.when(wid == NW - 1)
    def _():
        t0 = NW * CROWS
        pltpu.sync_copy(part_hbm.at[0].at[pl.ds(t0, TAIL_ROWS)],
                        bufa.at[pl.ds(0, TAIL_ROWS)])
        pltpu.sync_copy(part_hbm.at[1].at[pl.ds(t0, TAIL_ROWS)],
                        bufb.at[pl.ds(0, TAIL_ROWS)])
        add_rows(TAIL_ROWS)
        pltpu.sync_copy(bufa.at[pl.ds(0, TAIL_ROWS)],
                        h2_hbm.at[0].at[pl.ds(t0, TAIL_ROWS)])
        pltpu.sync_copy(bufa.at[pl.ds(0, TAIL_ROWS)],
                        h2_hbm.at[1].at[pl.ds(t0, TAIL_ROWS)])


def kernel(x, edge_row, edge_col, edge_vals):
    pad = E_PAD - N_EDGES
    edge_row = jnp.concatenate([edge_row, jnp.zeros((pad,), jnp.int32)])
    edge_col = jnp.concatenate([edge_col, jnp.zeros((pad,), jnp.int32)])
    edge_vals = jnp.concatenate([edge_vals, jnp.zeros((pad,), jnp.float32)])
    h2 = jnp.stack([x, x])
    for _ in range(NUM_LAYERS):
        part = _spmm(h2, edge_row, edge_col, edge_vals)
        h2 = _combine(part)
    return h2[0]


# speedup vs baseline: 1.8177x; 1.8177x over previous
import functools

import jax
import jax.numpy as jnp
from jax import lax
from jax.experimental import pallas as pl
from jax.experimental.pallas import tpu as pltpu
from jax.experimental.pallas import tpu_sc as plsc

N_NODES = 10000
N_EDGES = 320000
D_FEAT = 128
NUM_LAYERS = 3

NC = 2
NS = 16
NW = NC * NS
B = 128
NBF = 126
NBS = 33
E_PAD = NS * (NBF + NBS) * B
NBUF = 3
ROWS_PER_TILE = 624
TAIL_ROWS = N_NODES - NS * ROWS_PER_TILE

CROWS = 312

_MESH = plsc.VectorSubcoreMesh(
    core_axis_name="c", subcore_axis_name="s", num_cores=NC)


@functools.partial(
    pl.kernel,
    out_type=jax.ShapeDtypeStruct((NC, N_NODES, D_FEAT), jnp.float32),
    mesh=_MESH,
    scratch_types=[
        pltpu.VMEM_SHARED((N_NODES, D_FEAT), jnp.float32),
        [pltpu.VMEM((B,), jnp.int32)] * NBUF,
        [pltpu.VMEM((B,), jnp.int32)] * NBUF,
        [pltpu.VMEM((B,), jnp.float32)] * NBUF,
        [pltpu.VMEM((B, D_FEAT), jnp.float32)] * NBUF,
        [pltpu.SemaphoreType.DMA] * NBUF,
        [pltpu.SemaphoreType.DMA] * NBUF,
        [pltpu.SemaphoreType.DMA] * NBUF,
    ],
)
def _spmm(h2_hbm, rows_hbm, cols_hbm, vals_hbm, part_hbm,
          accum, colvs, rowvs, valvs, rbufs, isems, gsems, ssems):
    cid = lax.axis_index("c")
    sid = lax.axis_index("s")
    h_hbm = h2_hbm.at[cid]
    zero16 = jnp.zeros((16,), jnp.float32)

    def issue_idx(b, u, tile_base):
        base = (tile_base + b) * B
        pltpu.async_copy(cols_hbm.at[pl.ds(base, B)], colvs[u], isems[u])
        pltpu.async_copy(rows_hbm.at[pl.ds(base, B)], rowvs[u], isems[u])
        pltpu.async_copy(vals_hbm.at[pl.ds(base, B)], valvs[u], isems[u])

    def wait_idx(u):
        pltpu.make_async_copy(cols_hbm.at[pl.ds(0, B)], colvs[u],
                              isems[u]).wait()
        pltpu.make_async_copy(rows_hbm.at[pl.ds(0, B)], rowvs[u],
                              isems[u]).wait()
        pltpu.make_async_copy(vals_hbm.at[pl.ds(0, B)], valvs[u],
                              isems[u]).wait()

    def issue_gather(u):
        pltpu.async_copy(h_hbm.at[colvs[u]], rbufs[u], gsems[u])

    def wait_gather(u):
        pltpu.make_async_copy(h_hbm.at[colvs[u]], rbufs[u], gsems[u]).wait()

    def issue_scatter(u):
        pltpu.async_copy(rbufs[u], accum.at[rowvs[u]], ssems[u], add=True)

    def wait_scatter(u):
        pltpu.make_async_copy(rbufs[u], accum.at[rowvs[u]], ssems[u]).wait()

    def scale(u):
        buf = rbufs[u]
        valv = valvs[u]

        def group_body(g, _):
            vv = valv[pl.ds(g * 16, 16)]
            for t in range(16):
                v = vv[t]
                row = g * 16 + t
                for kk in range(D_FEAT // 16):
                    sl = pl.ds(kk * 16, 16)
                    buf[row, sl] = buf[row, sl] * v
            return 0
        lax.fori_loop(0, B // 16, group_body, 0)

    def zrow(i, _):
        for kk in range(D_FEAT // 16):
            rbufs[0][i, pl.ds(kk * 16, 16)] = zero16
        return 0
    lax.fori_loop(0, B, zrow, 0)
    for off, sz in [(0, B), (B, B), (2 * B, B), (3 * B, B), (4 * B, 112)]:
        r0 = pl.multiple_of(sid * ROWS_PER_TILE + off, 8)
        pltpu.sync_copy(rbufs[0].at[pl.ds(0, sz)], accum.at[pl.ds(r0, sz)])

    @pl.when(sid == NS - 1)
    def _():
        pltpu.sync_copy(rbufs[0].at[pl.ds(0, TAIL_ROWS)],
                        accum.at[pl.ds(NS * ROWS_PER_TILE, TAIL_ROWS)])

    plsc.subcore_barrier()

    def run_pipe(nb, tile_base):
        for u in range(NBUF):
            issue_idx(u, u, tile_base)
        for u in range(NBUF):
            wait_idx(u)
            issue_gather(u)

        def iter_body(i, _):
            for u in range(NBUF):
                b = i * NBUF + u
                prev = (u + NBUF - 1) % NBUF
                more = jnp.logical_and(b >= 1, b + NBUF - 1 < nb)
                wait_gather(u)

                @pl.when(more)
                def _():
                    wait_scatter(prev)
                    issue_idx(b + NBUF - 1, prev, tile_base)

                scale(u)
                issue_scatter(u)

                @pl.when(more)
                def _():
                    wait_idx(prev)
                    issue_gather(prev)
            return 0

        lax.fori_loop(0, nb // NBUF, iter_body, 0)
        for u in range(NBUF):
            wait_scatter(u)

    @pl.when(cid == 0)
    def _():
        run_pipe(NBF, sid * NBF)

    @pl.when(cid == 1)
    def _():
        run_pipe(NBS, NS * NBF + sid * NBS)

    plsc.subcore_barrier()

    for off, sz in [(0, B), (B, B), (2 * B, B), (3 * B, B), (4 * B, 112)]:
        r0 = pl.multiple_of(sid * ROWS_PER_TILE + off, 8)
        pltpu.sync_copy(accum.at[pl.ds(r0, sz)],
                        part_hbm.at[cid].at[pl.ds(r0, sz)])

    @pl.when(sid == NS - 1)
    def _():
        r0 = NS * ROWS_PER_TILE
        pltpu.sync_copy(accum.at[pl.ds(r0, TAIL_ROWS)],
                        part_hbm.at[cid].at[pl.ds(r0, TAIL_ROWS)])


@functools.partial(
    pl.kernel,
    out_type=jax.ShapeDtypeStruct((NC, N_NODES, D_FEAT), jnp.float32),
    mesh=_MESH,
    scratch_types=[
        pltpu.VMEM((CROWS, D_FEAT), jnp.float32),
        pltpu.VMEM((CROWS, D_FEAT), jnp.float32),
    ],
)
def _combine(part_hbm, h2_hbm, bufa, bufb):
    cid = lax.axis_index("c")
    sid = lax.axis_index("s")
    wid = cid * NS + sid
    r0 = pl.multiple_of(wid * CROWS, 8)

    def add_rows(nrows):
        def body(i, _):
            for kk in range(D_FEAT // 16):
                sl = pl.ds(kk * 16, 16)
                bufa[i, sl] = bufa[i, sl] + bufb[i, sl]
            return 0
        lax.fori_loop(0, nrows, body, 0)

    pltpu.sync_copy(part_hbm.at[0].at[pl.ds(r0, CROWS)], bufa)
    pltpu.sync_copy(part_hbm.at[1].at[pl.ds(r0, CROWS)], bufb)
    add_rows(CROWS)
    pltpu.sync_copy(bufa, h2_hbm.at[0].at[pl.ds(r0, CROWS)])
    pltpu.sync_copy(bufa, h2_hbm.at[1].at[pl.ds(r0, CROWS)])

    @pl.when(wid == NW - 1)
    def _():
        t0 = NW * CROWS
        pltpu.sync_copy(part_hbm.at[0].at[pl.ds(t0, TAIL_ROWS)],
                        bufa.at[pl.ds(0, TAIL_ROWS)])
        pltpu.sync_copy(part_hbm.at[1].at[pl.ds(t0, TAIL_ROWS)],
                        bufb.at[pl.ds(0, TAIL_ROWS)])
        add_rows(TAIL_ROWS)
        pltpu.sync_copy(bufa.at[pl.ds(0, TAIL_ROWS)],
                        h2_hbm.at[0].at[pl.ds(t0, TAIL_ROWS)])
        pltpu.sync_copy(bufa.at[pl.ds(0, TAIL_ROWS)],
                        h2_hbm.at[1].at[pl.ds(t0, TAIL_ROWS)])


def kernel(x, edge_row, edge_col, edge_vals):
    pad = E_PAD - N_EDGES
    edge_row = jnp.concatenate([edge_row, jnp.zeros((pad,), jnp.int32)])
    edge_col = jnp.concatenate([edge_col, jnp.zeros((pad,), jnp.int32)])
    edge_vals = jnp.concatenate([edge_vals, jnp.zeros((pad,), jnp.float32)])
    h2 = jnp.stack([x, x])
    for _ in range(NUM_LAYERS):
        part = _spmm(h2, edge_row, edge_col, edge_vals)
        h2 = _combine(part)
    return h2[0]


# speedup vs baseline: 1.8294x; 1.0064x over previous
import functools

import jax
import jax.numpy as jnp
from jax import lax
from jax.experimental import pallas as pl
from jax.experimental.pallas import tpu as pltpu
from jax.experimental.pallas import tpu_sc as plsc

N_NODES = 10000
N_EDGES = 320000
D_FEAT = 128
NUM_LAYERS = 3

NC = 2
NS = 16
NW = NC * NS
B = 128
NBF = 114
NBS = 45
E_PAD = NS * (NBF + NBS) * B
NBUF = 3
ROWS_PER_TILE = 624
TAIL_ROWS = N_NODES - NS * ROWS_PER_TILE

CROWS = 312

_MESH = plsc.VectorSubcoreMesh(
    core_axis_name="c", subcore_axis_name="s", num_cores=NC)


@functools.partial(
    pl.kernel,
    out_type=jax.ShapeDtypeStruct((NC, N_NODES, D_FEAT), jnp.float32),
    mesh=_MESH,
    scratch_types=[
        pltpu.VMEM_SHARED((N_NODES, D_FEAT), jnp.float32),
        [pltpu.VMEM((B,), jnp.int32)] * NBUF,
        [pltpu.VMEM((B,), jnp.int32)] * NBUF,
        [pltpu.VMEM((B,), jnp.float32)] * NBUF,
        [pltpu.VMEM((B, D_FEAT), jnp.float32)] * NBUF,
        [pltpu.SemaphoreType.DMA] * NBUF,
        [pltpu.SemaphoreType.DMA] * NBUF,
        [pltpu.SemaphoreType.DMA] * NBUF,
    ],
)
def _spmm(h2_hbm, rows_hbm, cols_hbm, vals_hbm, part_hbm,
          accum, colvs, rowvs, valvs, rbufs, isems, gsems, ssems):
    cid = lax.axis_index("c")
    sid = lax.axis_index("s")
    h_hbm = h2_hbm.at[cid]
    zero16 = jnp.zeros((16,), jnp.float32)

    def issue_idx(b, u, tile_base):
        base = (tile_base + b) * B
        pltpu.async_copy(cols_hbm.at[pl.ds(base, B)], colvs[u], isems[u])
        pltpu.async_copy(rows_hbm.at[pl.ds(base, B)], rowvs[u], isems[u])
        pltpu.async_copy(vals_hbm.at[pl.ds(base, B)], valvs[u], isems[u])

    def wait_idx(u):
        pltpu.make_async_copy(cols_hbm.at[pl.ds(0, B)], colvs[u],
                              isems[u]).wait()
        pltpu.make_async_copy(rows_hbm.at[pl.ds(0, B)], rowvs[u],
                              isems[u]).wait()
        pltpu.make_async_copy(vals_hbm.at[pl.ds(0, B)], valvs[u],
                              isems[u]).wait()

    def issue_gather(u):
        pltpu.async_copy(h_hbm.at[colvs[u]], rbufs[u], gsems[u])

    def wait_gather(u):
        pltpu.make_async_copy(h_hbm.at[colvs[u]], rbufs[u], gsems[u]).wait()

    def issue_scatter(u):
        pltpu.async_copy(rbufs[u], accum.at[rowvs[u]], ssems[u], add=True)

    def wait_scatter(u):
        pltpu.make_async_copy(rbufs[u], accum.at[rowvs[u]], ssems[u]).wait()

    def scale(u):
        buf = rbufs[u]
        valv = valvs[u]

        def group_body(g, _):
            vv = valv[pl.ds(g * 16, 16)]
            for t in range(16):
                v = vv[t]
                row = g * 16 + t
                for kk in range(D_FEAT // 16):
                    sl = pl.ds(kk * 16, 16)
                    buf[row, sl] = buf[row, sl] * v
            return 0
        lax.fori_loop(0, B // 16, group_body, 0)

    def zrow(i, _):
        for kk in range(D_FEAT // 16):
            rbufs[0][i, pl.ds(kk * 16, 16)] = zero16
        return 0
    lax.fori_loop(0, B, zrow, 0)
    for off, sz in [(0, B), (B, B), (2 * B, B), (3 * B, B), (4 * B, 112)]:
        r0 = pl.multiple_of(sid * ROWS_PER_TILE + off, 8)
        pltpu.sync_copy(rbufs[0].at[pl.ds(0, sz)], accum.at[pl.ds(r0, sz)])

    @pl.when(sid == NS - 1)
    def _():
        pltpu.sync_copy(rbufs[0].at[pl.ds(0, TAIL_ROWS)],
                        accum.at[pl.ds(NS * ROWS_PER_TILE, TAIL_ROWS)])

    plsc.subcore_barrier()

    def run_pipe(nb, tile_base):
        for u in range(NBUF):
            issue_idx(u, u, tile_base)
        for u in range(NBUF):
            wait_idx(u)
            issue_gather(u)

        def iter_body(i, _):
            for u in range(NBUF):
                b = i * NBUF + u
                prev = (u + NBUF - 1) % NBUF
                more = jnp.logical_and(b >= 1, b + NBUF - 1 < nb)
                wait_gather(u)

                @pl.when(more)
                def _():
                    wait_scatter(prev)
                    issue_idx(b + NBUF - 1, prev, tile_base)

                scale(u)
                issue_scatter(u)

                @pl.when(more)
                def _():
                    wait_idx(prev)
                    issue_gather(prev)
            return 0

        lax.fori_loop(0, nb // NBUF, iter_body, 0)
        for u in range(NBUF):
            wait_scatter(u)

    @pl.when(cid == 0)
    def _():
        run_pipe(NBF, sid * NBF)

    @pl.when(cid == 1)
    def _():
        run_pipe(NBS, NS * NBF + sid * NBS)

    plsc.subcore_barrier()

    for off, sz in [(0, B), (B, B), (2 * B, B), (3 * B, B), (4 * B, 112)]:
        r0 = pl.multiple_of(sid * ROWS_PER_TILE + off, 8)
        pltpu.sync_copy(accum.at[pl.ds(r0, sz)],
                        part_hbm.at[cid].at[pl.ds(r0, sz)])

    @pl.when(sid == NS - 1)
    def _():
        r0 = NS * ROWS_PER_TILE
        pltpu.sync_copy(accum.at[pl.ds(r0, TAIL_ROWS)],
                        part_hbm.at[cid].at[pl.ds(r0, TAIL_ROWS)])


@functools.partial(
    pl.kernel,
    out_type=jax.ShapeDtypeStruct((NC, N_NODES, D_FEAT), jnp.float32),
    mesh=_MESH,
    scratch_types=[
        pltpu.VMEM((CROWS, D_FEAT), jnp.float32),
        pltpu.VMEM((CROWS, D_FEAT), jnp.float32),
    ],
)
def _combine(part_hbm, h2_hbm, bufa, bufb):
    cid = lax.axis_index("c")
    sid = lax.axis_index("s")
    wid = cid * NS + sid
    r0 = pl.multiple_of(wid * CROWS, 8)

    def add_rows(nrows):
        def body(i, _):
            for kk in range(D_FEAT // 16):
                sl = pl.ds(kk * 16, 16)
                bufa[i, sl] = bufa[i, sl] + bufb[i, sl]
            return 0
        lax.fori_loop(0, nrows, body, 0)

    pltpu.sync_copy(part_hbm.at[0].at[pl.ds(r0, CROWS)], bufa)
    pltpu.sync_copy(part_hbm.at[1].at[pl.ds(r0, CROWS)], bufb)
    add_rows(CROWS)
    pltpu.sync_copy(bufa, h2_hbm.at[0].at[pl.ds(r0, CROWS)])
    pltpu.sync_copy(bufa, h2_hbm.at[1].at[pl.ds(r0, CROWS)])

    @pl.when(wid == NW - 1)
    def _():
        t0 = NW * CROWS
        pltpu.sync_copy(part_hbm.at[0].at[pl.ds(t0, TAIL_ROWS)],
                        bufa.at[pl.ds(0, TAIL_ROWS)])
        pltpu.sync_copy(part_hbm.at[1].at[pl.ds(t0, TAIL_ROWS)],
                        bufb.at[pl.ds(0, TAIL_ROWS)])
        add_rows(TAIL_ROWS)
        pltpu.sync_copy(bufa.at[pl.ds(0, TAIL_ROWS)],
                        h2_hbm.at[0].at[pl.ds(t0, TAIL_ROWS)])
        pltpu.sync_copy(bufa.at[pl.ds(0, TAIL_ROWS)],
                        h2_hbm.at[1].at[pl.ds(t0, TAIL_ROWS)])


def kernel(x, edge_row, edge_col, edge_vals):
    pad = E_PAD - N_EDGES
    edge_row = jnp.concatenate([edge_row, jnp.zeros((pad,), jnp.int32)])
    edge_col = jnp.concatenate([edge_col, jnp.zeros((pad,), jnp.int32)])
    edge_vals = jnp.concatenate([edge_vals, jnp.zeros((pad,), jnp.float32)])
    h2 = jnp.stack([x, x])
    for _ in range(NUM_LAYERS):
        part = _spmm(h2, edge_row, edge_col, edge_vals)
        h2 = _combine(part)
    return h2[0]
